# baseline (speedup 1.0000x reference)
import jax
import jax.numpy as jnp
from jax.experimental import pallas as pl
from jax.experimental.pallas import tpu as pltpu


def _body(tok_ref, tab_ref, out_ref):
    out_ref[...] = tok_ref[...] + tab_ref[...]


def kernel(encoded_tokens, pos_table):
    B, T, D = encoded_tokens.shape
    BT = 4096
    BB = 4

    return pl.pallas_call(
        _body,
        grid=(T // BT, B // BB),
        in_specs=[
            pl.BlockSpec((BB, BT, D), lambda t, b: (b, t, 0)),
            pl.BlockSpec((BT, D), lambda t, b: (t, 0)),
        ],
        out_specs=pl.BlockSpec((BB, BT, D), lambda t, b: (b, t, 0)),
        out_shape=jax.ShapeDtypeStruct((B, T, D), encoded_tokens.dtype),
        compiler_params=pltpu.CompilerParams(
            dimension_semantics=("arbitrary", "arbitrary"),
        ),
    )(encoded_tokens, pos_table)
